# Initial kernel scaffold; baseline (speedup 1.0000x reference)
import jax
import jax.numpy as jnp
from jax.experimental import pallas as pl


def kernel(S, M, instances, overlaps, W_center, W_radius, A_center, A_radius):
    raise NotImplementedError("write your pallas kernel here")


# speedup vs baseline: 29.9271x; 29.9271x over previous
import functools

import jax
import jax.numpy as jnp
from jax import lax
from jax.experimental import pallas as pl
from jax.experimental.pallas import tpu as pltpu
from jax.experimental.pallas import tpu_sc as plsc

_EPS = 1e-10
_DIM = 32
_SET_LEN = 32
_NPAIR = 16384
_NSETS_B = 2 * _NPAIR
_NIDS = _NSETS_B * _SET_LEN
_NW = 32
_P = 128
_NBLK = _NPAIR // _P

@functools.cache
def _sc_kernels():
    mesh = plsc.VectorSubcoreMesh(core_axis_name="c", subcore_axis_name="s")

    @functools.partial(
        pl.kernel,
        mesh=mesh,
        out_type=[
            jax.ShapeDtypeStruct((_NSETS_B, _SET_LEN), jnp.int32),
            jax.ShapeDtypeStruct((_NSETS_B, _SET_LEN), jnp.float32),
        ],
        scratch_types=[
            pltpu.VMEM((8, 128), jnp.int32),
            pltpu.VMEM((128, _SET_LEN), jnp.int32),
            pltpu.VMEM((128, _SET_LEN), jnp.float32),
            pltpu.SemaphoreType.DMA,
        ],
        compiler_params=pltpu.CompilerParams(use_tc_tiling_on_sc=False),
    )
    def _gather_sets(s_hbm, m_hbm, idx_hbm, items_out, mb_out, idx_v, s_v, m_v, sem):
        wid = lax.axis_index("s") * 2 + lax.axis_index("c")
        pltpu.sync_copy(idx_hbm.at[pl.ds(wid * 8, 8)], idx_v)
        for j in range(8):
            pltpu.async_copy(s_hbm.at[idx_v.at[j]], s_v, sem).wait()
            pltpu.sync_copy(s_v, items_out.at[pl.ds(wid * 1024 + j * 128, 128)])
            pltpu.async_copy(m_hbm.at[idx_v.at[j]], m_v, sem).wait()
            pltpu.sync_copy(m_v, mb_out.at[pl.ds(wid * 1024 + j * 128, 128)])

    @functools.partial(
        pl.kernel,
        mesh=mesh,
        out_type=[
            jax.ShapeDtypeStruct((_NIDS, _DIM), jnp.float32),
            jax.ShapeDtypeStruct((_NIDS, _DIM), jnp.float32),
        ],
        scratch_types=[
            pltpu.VMEM((8, 128), jnp.int32),
            pltpu.VMEM((1024, _DIM), jnp.float32),
            pltpu.VMEM((1024, _DIM), jnp.float32),
            pltpu.SemaphoreType.DMA,
        ],
        compiler_params=pltpu.CompilerParams(use_tc_tiling_on_sc=False),
    )
    def _gather_emb(wc_hbm, wr_hbm, ids_hbm, gc_out, gr_out, idx_v, c_v, r_v, sem):
        wid = lax.axis_index("s") * 2 + lax.axis_index("c")

        def body(c, carry):
            row0 = pl.multiple_of(wid * 256 + c * 8, 8)
            out0 = pl.multiple_of(wid * 32768 + c * 1024, 1024)
            pltpu.sync_copy(ids_hbm.at[pl.ds(row0, 8)], idx_v)
            cps = []
            for j in range(8):
                cps.append(
                    pltpu.async_copy(
                        wc_hbm.at[idx_v.at[j]], c_v.at[pl.ds(j * 128, 128)], sem
                    )
                )
                cps.append(
                    pltpu.async_copy(
                        wr_hbm.at[idx_v.at[j]], r_v.at[pl.ds(j * 128, 128)], sem
                    )
                )
            for cp in cps:
                cp.wait()
            pltpu.sync_copy(c_v, gc_out.at[pl.ds(out0, 1024)])
            pltpu.sync_copy(r_v, gr_out.at[pl.ds(out0, 1024)])
            return carry

        lax.fori_loop(0, 32, body, 0)

    return _gather_sets, _gather_emb


def _softplus(x):
    return jnp.maximum(x, 0.0) + jnp.log(1.0 + jnp.exp(-jnp.abs(x)))


def _attn_loss(xci_ref, xcj_ref, xri_ref, xrj_ref, mbi_ref, mbj_ref,
               ov_ref, ac_ref, ar_ref, out_ref):
    mask_i = mbi_ref[...] > 0.0
    mask_j = mbj_ref[...] > 0.0
    ac = ac_ref[0:1, :].reshape(1, 1, _DIM)
    ar = ar_ref[0:1, :].reshape(1, 1, _DIM)

    def pool(x, a_vec, mask, mrow, size_reg):
        att = jnp.sum(x * a_vec, axis=2)

        def segsoft(v):
            vm = jnp.where(mask, v, -jnp.inf)
            m = jnp.max(vm, axis=1, keepdims=True)
            w = jnp.where(mask, jnp.exp(v - m), 0.0)
            d = jnp.sum(w, axis=1, keepdims=True)
            return w / d

        w = segsoft(att)
        a = jnp.sum(x * w[:, :, None], axis=1)
        att2 = jnp.sum(x * a[:, None, :], axis=2)
        w2 = segsoft(att2)
        emb = jnp.sum(x * w2[:, :, None], axis=1)
        if size_reg:
            sizes = jnp.sum(mrow, axis=1, keepdims=True)
            emb = emb * jnp.exp(jnp.log(sizes) * (1.0 / _DIM))
        return emb

    c_i = pool(xci_ref[...], ac, mask_i, None, False)
    c_j = pool(xcj_ref[...], ac, mask_j, None, False)
    r_i = pool(xri_ref[...], ar, mask_i, mbi_ref[...], True)
    r_j = pool(xrj_ref[...], ar, mask_j, mbj_ref[...], True)

    m_i = _softplus(c_i)
    be_i = _softplus(r_i)
    Mi = m_i + be_i
    m_j = _softplus(c_j)
    be_j = _softplus(r_j)
    Mj = m_j + be_j
    delta = jnp.minimum(Mi, Mj) - jnp.maximum(m_i, m_j)
    bv_i = jnp.sum(jnp.log(be_i + _EPS), axis=1, keepdims=True)
    bv_j = jnp.sum(jnp.log(be_j + _EPS), axis=1, keepdims=True)
    inter = jnp.sum(jnp.log(delta + _EPS), axis=1, keepdims=True)
    union = jnp.sum(
        jnp.log(jnp.maximum(Mi, Mj) - jnp.minimum(m_i, m_j) + _EPS),
        axis=1, keepdims=True)
    c_overlap = inter
    c_jaccard = inter / union
    denom = jnp.log(jnp.abs(bv_i) + _EPS) + jnp.log(jnp.abs(bv_j) + _EPS)
    c_cosine = inter / jnp.exp(jnp.log(denom) * (1.0 / _DIM))
    c_dice = 2.0 * inter / (bv_i + bv_j)

    ov = ov_ref[...]
    d1 = jnp.exp(c_overlap) - ov[:, 0:1]
    d2 = jnp.exp(c_jaccard) - ov[:, 1:2]
    d3 = jnp.exp(c_cosine) - ov[:, 2:3]
    d4 = jnp.exp(c_dice) - ov[:, 3:4]
    parts = jnp.concatenate(
        [jnp.sum(d1 * d1, axis=0, keepdims=True),
         jnp.sum(d2 * d2, axis=0, keepdims=True),
         jnp.sum(d3 * d3, axis=0, keepdims=True),
         jnp.sum(d4 * d4, axis=0, keepdims=True)], axis=1)
    out_ref[...] = parts.reshape(1, 1, 4)


def kernel(S, M, instances, overlaps, W_center, W_radius, A_center, A_radius):
    gather_sets, gather_emb = _sc_kernels()
    flat2 = jnp.transpose(instances).reshape(_NSETS_B // 128, 128)
    items32, mb = gather_sets(S, M, flat2)
    ids2d = items32.reshape(_NIDS // 128, 128)
    gc, gr = gather_emb(W_center, W_radius, ids2d)
    gc3 = gc.reshape(_NSETS_B, _SET_LEN, _DIM)
    gr3 = gr.reshape(_NSETS_B, _SET_LEN, _DIM)
    ovt = jnp.transpose(overlaps)
    ac8 = jnp.broadcast_to(A_center, (8, _DIM))
    ar8 = jnp.broadcast_to(A_radius, (8, _DIM))

    parts = pl.pallas_call(
        _attn_loss,
        grid=(_NBLK,),
        in_specs=[
            pl.BlockSpec((_P, _SET_LEN, _DIM), lambda i: (i, 0, 0)),
            pl.BlockSpec((_P, _SET_LEN, _DIM), lambda i: (i + _NBLK, 0, 0)),
            pl.BlockSpec((_P, _SET_LEN, _DIM), lambda i: (i, 0, 0)),
            pl.BlockSpec((_P, _SET_LEN, _DIM), lambda i: (i + _NBLK, 0, 0)),
            pl.BlockSpec((_P, _SET_LEN), lambda i: (i, 0)),
            pl.BlockSpec((_P, _SET_LEN), lambda i: (i + _NBLK, 0)),
            pl.BlockSpec((_P, 4), lambda i: (i, 0)),
            pl.BlockSpec((8, _DIM), lambda i: (0, 0)),
            pl.BlockSpec((8, _DIM), lambda i: (0, 0)),
        ],
        out_specs=pl.BlockSpec((1, 1, 4), lambda i: (i, 0, 0)),
        out_shape=jax.ShapeDtypeStruct((_NBLK, 1, 4), jnp.float32),
    )(gc3, gc3, gr3, gr3, mb, mb, ovt, ac8, ar8)

    losses = jnp.sum(parts, axis=(0, 1))
    return (losses[0], losses[1], losses[2], losses[3])


# speedup vs baseline: 54.3110x; 1.8148x over previous
import functools

import jax
import jax.numpy as jnp
from jax import lax
from jax.experimental import pallas as pl
from jax.experimental.pallas import tpu as pltpu
from jax.experimental.pallas import tpu_sc as plsc

_EPS = 1e-10
_DIM = 32
_SET_LEN = 32
_NPAIR = 16384
_NSETS_B = 2 * _NPAIR
_NIDS = _NSETS_B * _SET_LEN
_NW = 32
_P = 128
_NBLK = _NPAIR // _P

@functools.cache
def _sc_kernels():
    mesh = plsc.VectorSubcoreMesh(core_axis_name="c", subcore_axis_name="s")

    @functools.partial(
        pl.kernel,
        mesh=mesh,
        out_type=[
            jax.ShapeDtypeStruct((_NSETS_B, _SET_LEN), jnp.int32),
            jax.ShapeDtypeStruct((_NSETS_B, _SET_LEN), jnp.float32),
        ],
        scratch_types=[
            pltpu.VMEM((8, 128), jnp.int32),
            pltpu.VMEM((128, _SET_LEN), jnp.int32),
            pltpu.VMEM((128, _SET_LEN), jnp.float32),
            pltpu.SemaphoreType.DMA,
        ],
        compiler_params=pltpu.CompilerParams(use_tc_tiling_on_sc=False),
    )
    def _gather_sets(s_hbm, m_hbm, idx_hbm, items_out, mb_out, idx_v, s_v, m_v, sem):
        wid = lax.axis_index("s") * 2 + lax.axis_index("c")
        pltpu.sync_copy(idx_hbm.at[pl.ds(wid * 8, 8)], idx_v)
        for j in range(8):
            pltpu.async_copy(s_hbm.at[idx_v.at[j]], s_v, sem).wait()
            pltpu.sync_copy(s_v, items_out.at[pl.ds(wid * 1024 + j * 128, 128)])
            pltpu.async_copy(m_hbm.at[idx_v.at[j]], m_v, sem).wait()
            pltpu.sync_copy(m_v, mb_out.at[pl.ds(wid * 1024 + j * 128, 128)])

    @functools.partial(
        pl.kernel,
        mesh=mesh,
        out_type=[
            jax.ShapeDtypeStruct((_NIDS, _DIM), jnp.float32),
            jax.ShapeDtypeStruct((_NIDS, _DIM), jnp.float32),
        ],
        scratch_types=[
            pltpu.VMEM((8, 128), jnp.int32),
            pltpu.VMEM((1024, _DIM), jnp.float32),
            pltpu.VMEM((1024, _DIM), jnp.float32),
            pltpu.SemaphoreType.DMA,
        ],
        compiler_params=pltpu.CompilerParams(use_tc_tiling_on_sc=False),
    )
    def _gather_emb(wc_hbm, wr_hbm, ids_hbm, gc_out, gr_out, idx_v, c_v, r_v, sem):
        wid = lax.axis_index("s") * 2 + lax.axis_index("c")

        def body(c, carry):
            row0 = pl.multiple_of(wid * 256 + c * 8, 8)
            out0 = pl.multiple_of(wid * 32768 + c * 1024, 1024)
            pltpu.sync_copy(ids_hbm.at[pl.ds(row0, 8)], idx_v)
            cps = []
            for j in range(8):
                cps.append(
                    pltpu.async_copy(
                        wc_hbm.at[idx_v.at[j]], c_v.at[pl.ds(j * 128, 128)], sem
                    )
                )
                cps.append(
                    pltpu.async_copy(
                        wr_hbm.at[idx_v.at[j]], r_v.at[pl.ds(j * 128, 128)], sem
                    )
                )
            for cp in cps:
                cp.wait()
            pltpu.sync_copy(c_v, gc_out.at[pl.ds(out0, 1024)])
            pltpu.sync_copy(r_v, gr_out.at[pl.ds(out0, 1024)])
            return carry

        lax.fori_loop(0, 32, body, 0)

    return _gather_sets, _gather_emb


def _softplus(x):
    return jnp.maximum(x, 0.0) + jnp.log(1.0 + jnp.exp(-jnp.abs(x)))


def _attn_loss(xci_ref, xcj_ref, xri_ref, xrj_ref, mbi_ref, mbj_ref,
               ov_ref, ac_ref, ar_ref, e_ref, d_ref, et_ref, dt_ref, out_ref):
    mask_i = mbi_ref[...] > 0.0
    mask_j = mbj_ref[...] > 0.0
    E = e_ref[...]
    D = d_ref[...]
    ET = et_ref[...]
    DT = dt_ref[...]
    ac = ac_ref[0:1, :]
    ar = ar_ref[0:1, :]

    def mm(a, b):
        return jax.lax.dot_general(
            a, b, (((1,), (0,)), ((), ())),
            preferred_element_type=jnp.float32)

    def pool(x, a_tile, mask, mrow, size_reg):
        att = mm(x * a_tile, E)

        def segsoft(v):
            vm = jnp.where(mask, v, -jnp.inf)
            m = jnp.max(vm, axis=1, keepdims=True)
            w = jnp.where(mask, jnp.exp(v - m), 0.0)
            d = jnp.sum(w, axis=1, keepdims=True)
            return w / d

        w = segsoft(att)
        a = mm(x * mm(w, ET), D)
        att2 = mm(x * mm(a, DT), E)
        w2 = segsoft(att2)
        emb = mm(x * mm(w2, ET), D)
        if size_reg:
            sizes = jnp.sum(mrow, axis=1, keepdims=True)
            emb = emb * jnp.exp(jnp.log(sizes) * (1.0 / _DIM))
        return emb

    c_i = pool(xci_ref[...], ac, mask_i, None, False)
    c_j = pool(xcj_ref[...], ac, mask_j, None, False)
    r_i = pool(xri_ref[...], ar, mask_i, mbi_ref[...], True)
    r_j = pool(xrj_ref[...], ar, mask_j, mbj_ref[...], True)

    m_i = _softplus(c_i)
    be_i = _softplus(r_i)
    Mi = m_i + be_i
    m_j = _softplus(c_j)
    be_j = _softplus(r_j)
    Mj = m_j + be_j
    delta = jnp.minimum(Mi, Mj) - jnp.maximum(m_i, m_j)
    bv_i = jnp.sum(jnp.log(be_i + _EPS), axis=1, keepdims=True)
    bv_j = jnp.sum(jnp.log(be_j + _EPS), axis=1, keepdims=True)
    inter = jnp.sum(jnp.log(delta + _EPS), axis=1, keepdims=True)
    union = jnp.sum(
        jnp.log(jnp.maximum(Mi, Mj) - jnp.minimum(m_i, m_j) + _EPS),
        axis=1, keepdims=True)
    c_overlap = inter
    c_jaccard = inter / union
    denom = jnp.log(jnp.abs(bv_i) + _EPS) + jnp.log(jnp.abs(bv_j) + _EPS)
    c_cosine = inter / jnp.exp(jnp.log(denom) * (1.0 / _DIM))
    c_dice = 2.0 * inter / (bv_i + bv_j)

    ov = ov_ref[...]
    d1 = jnp.exp(c_overlap) - ov[:, 0:1]
    d2 = jnp.exp(c_jaccard) - ov[:, 1:2]
    d3 = jnp.exp(c_cosine) - ov[:, 2:3]
    d4 = jnp.exp(c_dice) - ov[:, 3:4]
    parts = jnp.concatenate(
        [jnp.sum(d1 * d1, axis=0, keepdims=True),
         jnp.sum(d2 * d2, axis=0, keepdims=True),
         jnp.sum(d3 * d3, axis=0, keepdims=True),
         jnp.sum(d4 * d4, axis=0, keepdims=True)], axis=1)
    out_ref[...] = parts.reshape(1, 1, 4)


def kernel(S, M, instances, overlaps, W_center, W_radius, A_center, A_radius):
    gather_sets, gather_emb = _sc_kernels()
    flat2 = jnp.transpose(instances).reshape(_NSETS_B // 128, 128)
    items32, mb = gather_sets(S, M, flat2)
    ids2d = items32.reshape(_NIDS // 128, 128)
    gc, gr = gather_emb(W_center, W_radius, ids2d)
    F = _SET_LEN * _DIM
    gc2 = gc.reshape(_NSETS_B, F)
    gr2 = gr.reshape(_NSETS_B, F)
    ovt = jnp.transpose(overlaps)
    lanes = jnp.arange(F, dtype=jnp.int32)
    grp = jnp.arange(_SET_LEN, dtype=jnp.int32)
    E = ((lanes[:, None] // _DIM) == grp[None, :]).astype(jnp.float32)
    D = ((lanes[:, None] % _DIM) == grp[None, :]).astype(jnp.float32)
    ac8 = jnp.broadcast_to(jnp.tile(A_center, _SET_LEN), (8, F))
    ar8 = jnp.broadcast_to(jnp.tile(A_radius, _SET_LEN), (8, F))

    parts = pl.pallas_call(
        _attn_loss,
        grid=(_NBLK,),
        in_specs=[
            pl.BlockSpec((_P, F), lambda i: (i, 0)),
            pl.BlockSpec((_P, F), lambda i: (i + _NBLK, 0)),
            pl.BlockSpec((_P, F), lambda i: (i, 0)),
            pl.BlockSpec((_P, F), lambda i: (i + _NBLK, 0)),
            pl.BlockSpec((_P, _SET_LEN), lambda i: (i, 0)),
            pl.BlockSpec((_P, _SET_LEN), lambda i: (i + _NBLK, 0)),
            pl.BlockSpec((_P, 4), lambda i: (i, 0)),
            pl.BlockSpec((8, F), lambda i: (0, 0)),
            pl.BlockSpec((8, F), lambda i: (0, 0)),
            pl.BlockSpec((F, _SET_LEN), lambda i: (0, 0)),
            pl.BlockSpec((F, _SET_LEN), lambda i: (0, 0)),
            pl.BlockSpec((_SET_LEN, F), lambda i: (0, 0)),
            pl.BlockSpec((_SET_LEN, F), lambda i: (0, 0)),
        ],
        out_specs=pl.BlockSpec((1, 1, 4), lambda i: (i, 0, 0)),
        out_shape=jax.ShapeDtypeStruct((_NBLK, 1, 4), jnp.float32),
    )(gc2, gc2, gr2, gr2, mb, mb, ovt, ac8, ar8,
      E, D, jnp.transpose(E), jnp.transpose(D))

    losses = jnp.sum(parts, axis=(0, 1))
    return (losses[0], losses[1], losses[2], losses[3])


# speedup vs baseline: 59.8213x; 1.1015x over previous
import functools

import jax
import jax.numpy as jnp
from jax import lax
from jax.experimental import pallas as pl
from jax.experimental.pallas import tpu as pltpu
from jax.experimental.pallas import tpu_sc as plsc

_EPS = 1e-10
_DIM = 32
_SET_LEN = 32
_NPAIR = 16384
_NSETS_B = 2 * _NPAIR
_NIDS = _NSETS_B * _SET_LEN
_NW = 32
_P = 256
_NBLK = _NPAIR // _P

@functools.cache
def _sc_kernels():
    mesh = plsc.VectorSubcoreMesh(core_axis_name="c", subcore_axis_name="s")

    @functools.partial(
        pl.kernel,
        mesh=mesh,
        out_type=[
            jax.ShapeDtypeStruct((_NSETS_B, _SET_LEN), jnp.int32),
            jax.ShapeDtypeStruct((_NSETS_B, _SET_LEN), jnp.float32),
        ],
        scratch_types=[
            pltpu.VMEM((8, 128), jnp.int32),
            pltpu.VMEM((128, _SET_LEN), jnp.int32),
            pltpu.VMEM((128, _SET_LEN), jnp.float32),
            pltpu.SemaphoreType.DMA,
        ],
        compiler_params=pltpu.CompilerParams(use_tc_tiling_on_sc=False),
    )
    def _gather_sets(s_hbm, m_hbm, idx_hbm, items_out, mb_out, idx_v, s_v, m_v, sem):
        wid = lax.axis_index("s") * 2 + lax.axis_index("c")
        pltpu.sync_copy(idx_hbm.at[pl.ds(wid * 8, 8)], idx_v)
        for j in range(8):
            pltpu.async_copy(s_hbm.at[idx_v.at[j]], s_v, sem).wait()
            pltpu.sync_copy(s_v, items_out.at[pl.ds(wid * 1024 + j * 128, 128)])
            pltpu.async_copy(m_hbm.at[idx_v.at[j]], m_v, sem).wait()
            pltpu.sync_copy(m_v, mb_out.at[pl.ds(wid * 1024 + j * 128, 128)])

    @functools.partial(
        pl.kernel,
        mesh=mesh,
        out_type=[
            jax.ShapeDtypeStruct((_NIDS, _DIM), jnp.float32),
            jax.ShapeDtypeStruct((_NIDS, _DIM), jnp.float32),
        ],
        scratch_types=[
            pltpu.VMEM((8, 128), jnp.int32),
            pltpu.VMEM((1024, _DIM), jnp.float32),
            pltpu.VMEM((1024, _DIM), jnp.float32),
            pltpu.SemaphoreType.DMA,
        ],
        compiler_params=pltpu.CompilerParams(use_tc_tiling_on_sc=False),
    )
    def _gather_emb(wc_hbm, wr_hbm, ids_hbm, gc_out, gr_out, idx_v, c_v, r_v, sem):
        wid = lax.axis_index("s") * 2 + lax.axis_index("c")

        def body(c, carry):
            row0 = pl.multiple_of(wid * 256 + c * 8, 8)
            out0 = pl.multiple_of(wid * 32768 + c * 1024, 1024)
            pltpu.sync_copy(ids_hbm.at[pl.ds(row0, 8)], idx_v)
            cps = []
            for j in range(8):
                cps.append(
                    pltpu.async_copy(
                        wc_hbm.at[idx_v.at[j]], c_v.at[pl.ds(j * 128, 128)], sem
                    )
                )
                cps.append(
                    pltpu.async_copy(
                        wr_hbm.at[idx_v.at[j]], r_v.at[pl.ds(j * 128, 128)], sem
                    )
                )
            for cp in cps:
                cp.wait()
            pltpu.sync_copy(c_v, gc_out.at[pl.ds(out0, 1024)])
            pltpu.sync_copy(r_v, gr_out.at[pl.ds(out0, 1024)])
            return carry

        lax.fori_loop(0, 32, body, 0)

    return _gather_sets, _gather_emb


def _softplus(x):
    return jnp.maximum(x, 0.0) + jnp.log(1.0 + jnp.exp(-jnp.abs(x)))


def _attn_loss(xci_ref, xcj_ref, xri_ref, xrj_ref, mbi_ref, mbj_ref,
               ov_ref, ac_ref, ar_ref, e_ref, d_ref, et_ref, dt_ref, out_ref):
    mask_i = mbi_ref[...] > 0.0
    mask_j = mbj_ref[...] > 0.0
    E = e_ref[...]
    D = d_ref[...]
    ET = et_ref[...]
    DT = dt_ref[...]
    ac = ac_ref[0:1, :]
    ar = ar_ref[0:1, :]

    def mm(a, b):
        return jax.lax.dot_general(
            a, b, (((1,), (0,)), ((), ())),
            preferred_element_type=jnp.float32)

    def pool(x, a_tile, mask, mrow, size_reg):
        att = mm(x * a_tile, E)

        def segsoft(v):
            vm = jnp.where(mask, v, -jnp.inf)
            m = jnp.max(vm, axis=1, keepdims=True)
            w = jnp.where(mask, jnp.exp(v - m), 0.0)
            d = jnp.sum(w, axis=1, keepdims=True)
            return w / d

        w = segsoft(att)
        a = mm(x * mm(w, ET), D)
        att2 = mm(x * mm(a, DT), E)
        w2 = segsoft(att2)
        emb = mm(x * mm(w2, ET), D)
        if size_reg:
            sizes = jnp.sum(mrow, axis=1, keepdims=True)
            emb = emb * jnp.exp(jnp.log(sizes) * (1.0 / _DIM))
        return emb

    c_i = pool(xci_ref[...], ac, mask_i, None, False)
    c_j = pool(xcj_ref[...], ac, mask_j, None, False)
    r_i = pool(xri_ref[...], ar, mask_i, mbi_ref[...], True)
    r_j = pool(xrj_ref[...], ar, mask_j, mbj_ref[...], True)

    m_i = _softplus(c_i)
    be_i = _softplus(r_i)
    Mi = m_i + be_i
    m_j = _softplus(c_j)
    be_j = _softplus(r_j)
    Mj = m_j + be_j
    delta = jnp.minimum(Mi, Mj) - jnp.maximum(m_i, m_j)
    bv_i = jnp.sum(jnp.log(be_i + _EPS), axis=1, keepdims=True)
    bv_j = jnp.sum(jnp.log(be_j + _EPS), axis=1, keepdims=True)
    inter = jnp.sum(jnp.log(delta + _EPS), axis=1, keepdims=True)
    union = jnp.sum(
        jnp.log(jnp.maximum(Mi, Mj) - jnp.minimum(m_i, m_j) + _EPS),
        axis=1, keepdims=True)
    c_overlap = inter
    c_jaccard = inter / union
    denom = jnp.log(jnp.abs(bv_i) + _EPS) + jnp.log(jnp.abs(bv_j) + _EPS)
    c_cosine = inter / jnp.exp(jnp.log(denom) * (1.0 / _DIM))
    c_dice = 2.0 * inter / (bv_i + bv_j)

    ov = ov_ref[...]
    d1 = jnp.exp(c_overlap) - ov[:, 0:1]
    d2 = jnp.exp(c_jaccard) - ov[:, 1:2]
    d3 = jnp.exp(c_cosine) - ov[:, 2:3]
    d4 = jnp.exp(c_dice) - ov[:, 3:4]
    parts = jnp.concatenate(
        [jnp.sum(d1 * d1, axis=0, keepdims=True),
         jnp.sum(d2 * d2, axis=0, keepdims=True),
         jnp.sum(d3 * d3, axis=0, keepdims=True),
         jnp.sum(d4 * d4, axis=0, keepdims=True)], axis=1)
    out_ref[...] = parts.reshape(1, 1, 4)


def kernel(S, M, instances, overlaps, W_center, W_radius, A_center, A_radius):
    gather_sets, gather_emb = _sc_kernels()
    flat2 = jnp.transpose(instances).reshape(_NSETS_B // 128, 128)
    items32, mb = gather_sets(S, M, flat2)
    ids2d = items32.reshape(_NIDS // 128, 128)
    gc, gr = gather_emb(W_center, W_radius, ids2d)
    F = _SET_LEN * _DIM
    gc2 = gc.reshape(_NSETS_B, F)
    gr2 = gr.reshape(_NSETS_B, F)
    ovt = jnp.transpose(overlaps)
    lanes = jnp.arange(F, dtype=jnp.int32)
    grp = jnp.arange(_SET_LEN, dtype=jnp.int32)
    E = ((lanes[:, None] // _DIM) == grp[None, :]).astype(jnp.float32)
    D = ((lanes[:, None] % _DIM) == grp[None, :]).astype(jnp.float32)
    ac8 = jnp.broadcast_to(jnp.tile(A_center, _SET_LEN), (8, F))
    ar8 = jnp.broadcast_to(jnp.tile(A_radius, _SET_LEN), (8, F))

    parts = pl.pallas_call(
        _attn_loss,
        grid=(_NBLK,),
        in_specs=[
            pl.BlockSpec((_P, F), lambda i: (i, 0)),
            pl.BlockSpec((_P, F), lambda i: (i + _NBLK, 0)),
            pl.BlockSpec((_P, F), lambda i: (i, 0)),
            pl.BlockSpec((_P, F), lambda i: (i + _NBLK, 0)),
            pl.BlockSpec((_P, _SET_LEN), lambda i: (i, 0)),
            pl.BlockSpec((_P, _SET_LEN), lambda i: (i + _NBLK, 0)),
            pl.BlockSpec((_P, 4), lambda i: (i, 0)),
            pl.BlockSpec((8, F), lambda i: (0, 0)),
            pl.BlockSpec((8, F), lambda i: (0, 0)),
            pl.BlockSpec((F, _SET_LEN), lambda i: (0, 0)),
            pl.BlockSpec((F, _SET_LEN), lambda i: (0, 0)),
            pl.BlockSpec((_SET_LEN, F), lambda i: (0, 0)),
            pl.BlockSpec((_SET_LEN, F), lambda i: (0, 0)),
        ],
        out_specs=pl.BlockSpec((1, 1, 4), lambda i: (i, 0, 0)),
        out_shape=jax.ShapeDtypeStruct((_NBLK, 1, 4), jnp.float32),
    )(gc2, gc2, gr2, gr2, mb, mb, ovt, ac8, ar8,
      E, D, jnp.transpose(E), jnp.transpose(D))

    losses = jnp.sum(parts, axis=(0, 1))
    return (losses[0], losses[1], losses[2], losses[3])


# speedup vs baseline: 63.8443x; 1.0673x over previous
import functools

import jax
import jax.numpy as jnp
from jax import lax
from jax.experimental import pallas as pl
from jax.experimental.pallas import tpu as pltpu
from jax.experimental.pallas import tpu_sc as plsc

_EPS = 1e-10
_DIM = 32
_SET_LEN = 32
_NPAIR = 16384
_NSETS_B = 2 * _NPAIR
_NIDS = _NSETS_B * _SET_LEN
_NW = 32
_P = 512
_NBLK = _NPAIR // _P

@functools.cache
def _sc_kernels():
    mesh = plsc.VectorSubcoreMesh(core_axis_name="c", subcore_axis_name="s")

    @functools.partial(
        pl.kernel,
        mesh=mesh,
        out_type=[
            jax.ShapeDtypeStruct((_NSETS_B, _SET_LEN), jnp.int32),
            jax.ShapeDtypeStruct((_NSETS_B, _SET_LEN), jnp.float32),
        ],
        scratch_types=[
            pltpu.VMEM((8, 128), jnp.int32),
            pltpu.VMEM((128, _SET_LEN), jnp.int32),
            pltpu.VMEM((128, _SET_LEN), jnp.float32),
            pltpu.SemaphoreType.DMA,
        ],
        compiler_params=pltpu.CompilerParams(use_tc_tiling_on_sc=False),
    )
    def _gather_sets(s_hbm, m_hbm, idx_hbm, items_out, mb_out, idx_v, s_v, m_v, sem):
        wid = lax.axis_index("s") * 2 + lax.axis_index("c")
        pltpu.sync_copy(idx_hbm.at[pl.ds(wid * 8, 8)], idx_v)
        for j in range(8):
            pltpu.async_copy(s_hbm.at[idx_v.at[j]], s_v, sem).wait()
            pltpu.sync_copy(s_v, items_out.at[pl.ds(wid * 1024 + j * 128, 128)])
            pltpu.async_copy(m_hbm.at[idx_v.at[j]], m_v, sem).wait()
            pltpu.sync_copy(m_v, mb_out.at[pl.ds(wid * 1024 + j * 128, 128)])

    @functools.partial(
        pl.kernel,
        mesh=mesh,
        out_type=[
            jax.ShapeDtypeStruct((_NIDS, _DIM), jnp.float32),
            jax.ShapeDtypeStruct((_NIDS, _DIM), jnp.float32),
        ],
        scratch_types=[
            pltpu.VMEM((8, 128), jnp.int32),
            pltpu.VMEM((1024, _DIM), jnp.float32),
            pltpu.VMEM((1024, _DIM), jnp.float32),
            pltpu.SemaphoreType.DMA,
        ],
        compiler_params=pltpu.CompilerParams(use_tc_tiling_on_sc=False),
    )
    def _gather_emb(wc_hbm, wr_hbm, ids_hbm, gc_out, gr_out, idx_v, c_v, r_v, sem):
        wid = lax.axis_index("s") * 2 + lax.axis_index("c")

        def body(c, carry):
            row0 = pl.multiple_of(wid * 256 + c * 8, 8)
            out0 = pl.multiple_of(wid * 32768 + c * 1024, 1024)
            pltpu.sync_copy(ids_hbm.at[pl.ds(row0, 8)], idx_v)
            cps = []
            for j in range(8):
                cps.append(
                    pltpu.async_copy(
                        wc_hbm.at[idx_v.at[j]], c_v.at[pl.ds(j * 128, 128)], sem
                    )
                )
                cps.append(
                    pltpu.async_copy(
                        wr_hbm.at[idx_v.at[j]], r_v.at[pl.ds(j * 128, 128)], sem
                    )
                )
            for cp in cps:
                cp.wait()
            pltpu.sync_copy(c_v, gc_out.at[pl.ds(out0, 1024)])
            pltpu.sync_copy(r_v, gr_out.at[pl.ds(out0, 1024)])
            return carry

        lax.fori_loop(0, 32, body, 0)

    return _gather_sets, _gather_emb


def _softplus(x):
    return jnp.maximum(x, 0.0) + jnp.log(1.0 + jnp.exp(-jnp.abs(x)))


def _attn_loss(xci_ref, xcj_ref, xri_ref, xrj_ref, mbi_ref, mbj_ref,
               ov_ref, ac_ref, ar_ref, e_ref, d_ref, et_ref, dt_ref, out_ref):
    mask_i = mbi_ref[...] > 0.0
    mask_j = mbj_ref[...] > 0.0
    E = e_ref[...]
    D = d_ref[...]
    ET = et_ref[...]
    DT = dt_ref[...]
    ac = ac_ref[0:1, :]
    ar = ar_ref[0:1, :]

    def mm(a, b):
        return jax.lax.dot_general(
            a, b, (((1,), (0,)), ((), ())),
            preferred_element_type=jnp.float32)

    def pool(x, a_tile, mask, mrow, size_reg):
        att = mm(x * a_tile, E)

        def segsoft(v):
            vm = jnp.where(mask, v, -jnp.inf)
            m = jnp.max(vm, axis=1, keepdims=True)
            w = jnp.where(mask, jnp.exp(v - m), 0.0)
            d = jnp.sum(w, axis=1, keepdims=True)
            return w / d

        w = segsoft(att)
        a = mm(x * mm(w, ET), D)
        att2 = mm(x * mm(a, DT), E)
        w2 = segsoft(att2)
        emb = mm(x * mm(w2, ET), D)
        if size_reg:
            sizes = jnp.sum(mrow, axis=1, keepdims=True)
            emb = emb * jnp.exp(jnp.log(sizes) * (1.0 / _DIM))
        return emb

    c_i = pool(xci_ref[...], ac, mask_i, None, False)
    c_j = pool(xcj_ref[...], ac, mask_j, None, False)
    r_i = pool(xri_ref[...], ar, mask_i, mbi_ref[...], True)
    r_j = pool(xrj_ref[...], ar, mask_j, mbj_ref[...], True)

    m_i = _softplus(c_i)
    be_i = _softplus(r_i)
    Mi = m_i + be_i
    m_j = _softplus(c_j)
    be_j = _softplus(r_j)
    Mj = m_j + be_j
    delta = jnp.minimum(Mi, Mj) - jnp.maximum(m_i, m_j)
    bv_i = jnp.sum(jnp.log(be_i + _EPS), axis=1, keepdims=True)
    bv_j = jnp.sum(jnp.log(be_j + _EPS), axis=1, keepdims=True)
    inter = jnp.sum(jnp.log(delta + _EPS), axis=1, keepdims=True)
    union = jnp.sum(
        jnp.log(jnp.maximum(Mi, Mj) - jnp.minimum(m_i, m_j) + _EPS),
        axis=1, keepdims=True)
    c_overlap = inter
    c_jaccard = inter / union
    denom = jnp.log(jnp.abs(bv_i) + _EPS) + jnp.log(jnp.abs(bv_j) + _EPS)
    c_cosine = inter / jnp.exp(jnp.log(denom) * (1.0 / _DIM))
    c_dice = 2.0 * inter / (bv_i + bv_j)

    ov = ov_ref[...]
    d1 = jnp.exp(c_overlap) - ov[:, 0:1]
    d2 = jnp.exp(c_jaccard) - ov[:, 1:2]
    d3 = jnp.exp(c_cosine) - ov[:, 2:3]
    d4 = jnp.exp(c_dice) - ov[:, 3:4]
    parts = jnp.concatenate(
        [jnp.sum(d1 * d1, axis=0, keepdims=True),
         jnp.sum(d2 * d2, axis=0, keepdims=True),
         jnp.sum(d3 * d3, axis=0, keepdims=True),
         jnp.sum(d4 * d4, axis=0, keepdims=True)], axis=1)
    out_ref[...] = parts.reshape(1, 1, 4)


def kernel(S, M, instances, overlaps, W_center, W_radius, A_center, A_radius):
    gather_sets, gather_emb = _sc_kernels()
    flat2 = jnp.transpose(instances).reshape(_NSETS_B // 128, 128)
    items32, mb = gather_sets(S, M, flat2)
    ids2d = items32.reshape(_NIDS // 128, 128)
    gc, gr = gather_emb(W_center, W_radius, ids2d)
    F = _SET_LEN * _DIM
    gc2 = gc.reshape(_NSETS_B, F)
    gr2 = gr.reshape(_NSETS_B, F)
    ovt = jnp.transpose(overlaps)
    lanes = jnp.arange(F, dtype=jnp.int32)
    grp = jnp.arange(_SET_LEN, dtype=jnp.int32)
    E = ((lanes[:, None] // _DIM) == grp[None, :]).astype(jnp.float32)
    D = ((lanes[:, None] % _DIM) == grp[None, :]).astype(jnp.float32)
    ac8 = jnp.broadcast_to(jnp.tile(A_center, _SET_LEN), (8, F))
    ar8 = jnp.broadcast_to(jnp.tile(A_radius, _SET_LEN), (8, F))

    parts = pl.pallas_call(
        _attn_loss,
        grid=(_NBLK,),
        in_specs=[
            pl.BlockSpec((_P, F), lambda i: (i, 0)),
            pl.BlockSpec((_P, F), lambda i: (i + _NBLK, 0)),
            pl.BlockSpec((_P, F), lambda i: (i, 0)),
            pl.BlockSpec((_P, F), lambda i: (i + _NBLK, 0)),
            pl.BlockSpec((_P, _SET_LEN), lambda i: (i, 0)),
            pl.BlockSpec((_P, _SET_LEN), lambda i: (i + _NBLK, 0)),
            pl.BlockSpec((_P, 4), lambda i: (i, 0)),
            pl.BlockSpec((8, F), lambda i: (0, 0)),
            pl.BlockSpec((8, F), lambda i: (0, 0)),
            pl.BlockSpec((F, _SET_LEN), lambda i: (0, 0)),
            pl.BlockSpec((F, _SET_LEN), lambda i: (0, 0)),
            pl.BlockSpec((_SET_LEN, F), lambda i: (0, 0)),
            pl.BlockSpec((_SET_LEN, F), lambda i: (0, 0)),
        ],
        out_specs=pl.BlockSpec((1, 1, 4), lambda i: (i, 0, 0)),
        out_shape=jax.ShapeDtypeStruct((_NBLK, 1, 4), jnp.float32),
    )(gc2, gc2, gr2, gr2, mb, mb, ovt, ac8, ar8,
      E, D, jnp.transpose(E), jnp.transpose(D))

    losses = jnp.sum(parts, axis=(0, 1))
    return (losses[0], losses[1], losses[2], losses[3])


# speedup vs baseline: 64.1436x; 1.0047x over previous
import functools

import jax
import jax.numpy as jnp
from jax import lax
from jax.experimental import pallas as pl
from jax.experimental.pallas import tpu as pltpu
from jax.experimental.pallas import tpu_sc as plsc

_EPS = 1e-10
_DIM = 32
_SET_LEN = 32
_NPAIR = 16384
_NSETS_B = 2 * _NPAIR
_NIDS = _NSETS_B * _SET_LEN
_NW = 32
_P = 512
_NBLK = _NPAIR // _P

@functools.cache
def _sc_kernels():
    mesh = plsc.VectorSubcoreMesh(core_axis_name="c", subcore_axis_name="s")

    @functools.partial(
        pl.kernel,
        mesh=mesh,
        out_type=[
            jax.ShapeDtypeStruct((_NSETS_B, _SET_LEN), jnp.int32),
            jax.ShapeDtypeStruct((_NSETS_B, _SET_LEN), jnp.float32),
        ],
        scratch_types=[
            pltpu.VMEM((8, 128), jnp.int32),
            pltpu.VMEM((128, _SET_LEN), jnp.int32),
            pltpu.VMEM((128, _SET_LEN), jnp.float32),
            pltpu.SemaphoreType.DMA,
        ],
        compiler_params=pltpu.CompilerParams(use_tc_tiling_on_sc=False),
    )
    def _gather_sets(s_hbm, m_hbm, idx_hbm, items_out, mb_out, idx_v, s_v, m_v, sem):
        wid = lax.axis_index("s") * 2 + lax.axis_index("c")
        pltpu.sync_copy(idx_hbm.at[pl.ds(wid * 8, 8)], idx_v)
        for j in range(8):
            pltpu.async_copy(s_hbm.at[idx_v.at[j]], s_v, sem).wait()
            pltpu.sync_copy(s_v, items_out.at[pl.ds(wid * 1024 + j * 128, 128)])
            pltpu.async_copy(m_hbm.at[idx_v.at[j]], m_v, sem).wait()
            pltpu.sync_copy(m_v, mb_out.at[pl.ds(wid * 1024 + j * 128, 128)])

    @functools.partial(
        pl.kernel,
        mesh=mesh,
        out_type=[
            jax.ShapeDtypeStruct((_NIDS, _DIM), jnp.float32),
            jax.ShapeDtypeStruct((_NIDS, _DIM), jnp.float32),
        ],
        scratch_types=[
            pltpu.VMEM((4, 128), jnp.int32),
            pltpu.VMEM((512, _DIM), jnp.float32),
            pltpu.VMEM((512, _DIM), jnp.float32),
            pltpu.VMEM((512, _DIM), jnp.float32),
            pltpu.VMEM((512, _DIM), jnp.float32),
            pltpu.SemaphoreType.DMA,
            pltpu.SemaphoreType.DMA,
            pltpu.SemaphoreType.DMA,
        ],
        compiler_params=pltpu.CompilerParams(use_tc_tiling_on_sc=False),
    )
    def _gather_emb(wc_hbm, wr_hbm, ids_hbm, gc_out, gr_out,
                    idx_v, c_v, r_v, c2_v, r2_v, sem, semw, semw2):
        wid = lax.axis_index("s") * 2 + lax.axis_index("c")

        def chunk(c, cbuf, rbuf, first, semw):
            row0 = pl.multiple_of(wid * 256 + c * 4, 4)
            out0 = pl.multiple_of(wid * 32768 + c * 512, 512)
            pltpu.sync_copy(ids_hbm.at[pl.ds(row0, 4)], idx_v)
            @pl.when(jnp.logical_not(first))
            def _():
                pltpu.make_async_copy(cbuf, gc_out.at[pl.ds(out0, 512)], semw).wait()
                pltpu.make_async_copy(rbuf, gr_out.at[pl.ds(out0, 512)], semw).wait()

            cps = []
            for j in range(4):
                cps.append(
                    pltpu.async_copy(
                        wc_hbm.at[idx_v.at[j]], cbuf.at[pl.ds(j * 128, 128)], sem
                    )
                )
                cps.append(
                    pltpu.async_copy(
                        wr_hbm.at[idx_v.at[j]], rbuf.at[pl.ds(j * 128, 128)], sem
                    )
                )
            for cp in cps:
                cp.wait()
            pltpu.async_copy(cbuf, gc_out.at[pl.ds(out0, 512)], semw)
            pltpu.async_copy(rbuf, gr_out.at[pl.ds(out0, 512)], semw)

        def body(g, carry):
            chunk(2 * g, c_v, r_v, g == 0, semw)
            chunk(2 * g + 1, c2_v, r2_v, g == 0, semw2)
            return carry

        lax.fori_loop(0, 32, body, 0)
        pltpu.make_async_copy(c_v, gc_out.at[pl.ds(0, 512)], semw).wait()
        pltpu.make_async_copy(r_v, gr_out.at[pl.ds(0, 512)], semw).wait()
        pltpu.make_async_copy(c2_v, gc_out.at[pl.ds(0, 512)], semw2).wait()
        pltpu.make_async_copy(r2_v, gr_out.at[pl.ds(0, 512)], semw2).wait()

    return _gather_sets, _gather_emb


def _softplus(x):
    return jnp.maximum(x, 0.0) + jnp.log(1.0 + jnp.exp(-jnp.abs(x)))


def _attn_loss(xci_ref, xcj_ref, xri_ref, xrj_ref, mbi_ref, mbj_ref,
               ov_ref, ac_ref, ar_ref, e_ref, d_ref, et_ref, dt_ref, out_ref):
    mask_i = mbi_ref[...] > 0.0
    mask_j = mbj_ref[...] > 0.0
    E = e_ref[...]
    D = d_ref[...]
    ET = et_ref[...]
    DT = dt_ref[...]
    ac = ac_ref[0:1, :]
    ar = ar_ref[0:1, :]

    def mm(a, b):
        return jax.lax.dot_general(
            a, b, (((1,), (0,)), ((), ())),
            preferred_element_type=jnp.float32)

    def pool(x, a_tile, mask, mrow, size_reg):
        att = mm(x * a_tile, E)

        def segsoft(v):
            vm = jnp.where(mask, v, -jnp.inf)
            m = jnp.max(vm, axis=1, keepdims=True)
            w = jnp.where(mask, jnp.exp(v - m), 0.0)
            d = jnp.sum(w, axis=1, keepdims=True)
            return w / d

        w = segsoft(att)
        a = mm(x * mm(w, ET), D)
        att2 = mm(x * mm(a, DT), E)
        w2 = segsoft(att2)
        emb = mm(x * mm(w2, ET), D)
        if size_reg:
            sizes = jnp.sum(mrow, axis=1, keepdims=True)
            emb = emb * jnp.exp(jnp.log(sizes) * (1.0 / _DIM))
        return emb

    c_i = pool(xci_ref[...], ac, mask_i, None, False)
    c_j = pool(xcj_ref[...], ac, mask_j, None, False)
    r_i = pool(xri_ref[...], ar, mask_i, mbi_ref[...], True)
    r_j = pool(xrj_ref[...], ar, mask_j, mbj_ref[...], True)

    m_i = _softplus(c_i)
    be_i = _softplus(r_i)
    Mi = m_i + be_i
    m_j = _softplus(c_j)
    be_j = _softplus(r_j)
    Mj = m_j + be_j
    delta = jnp.minimum(Mi, Mj) - jnp.maximum(m_i, m_j)
    bv_i = jnp.sum(jnp.log(be_i + _EPS), axis=1, keepdims=True)
    bv_j = jnp.sum(jnp.log(be_j + _EPS), axis=1, keepdims=True)
    inter = jnp.sum(jnp.log(delta + _EPS), axis=1, keepdims=True)
    union = jnp.sum(
        jnp.log(jnp.maximum(Mi, Mj) - jnp.minimum(m_i, m_j) + _EPS),
        axis=1, keepdims=True)
    c_overlap = inter
    c_jaccard = inter / union
    denom = jnp.log(jnp.abs(bv_i) + _EPS) + jnp.log(jnp.abs(bv_j) + _EPS)
    c_cosine = inter / jnp.exp(jnp.log(denom) * (1.0 / _DIM))
    c_dice = 2.0 * inter / (bv_i + bv_j)

    ov = ov_ref[...]
    d1 = jnp.exp(c_overlap) - ov[:, 0:1]
    d2 = jnp.exp(c_jaccard) - ov[:, 1:2]
    d3 = jnp.exp(c_cosine) - ov[:, 2:3]
    d4 = jnp.exp(c_dice) - ov[:, 3:4]
    parts = jnp.concatenate(
        [jnp.sum(d1 * d1, axis=0, keepdims=True),
         jnp.sum(d2 * d2, axis=0, keepdims=True),
         jnp.sum(d3 * d3, axis=0, keepdims=True),
         jnp.sum(d4 * d4, axis=0, keepdims=True)], axis=1)
    out_ref[...] = parts.reshape(1, 1, 4)


def kernel(S, M, instances, overlaps, W_center, W_radius, A_center, A_radius):
    gather_sets, gather_emb = _sc_kernels()
    flat2 = jnp.transpose(instances).reshape(_NSETS_B // 128, 128)
    items32, mb = gather_sets(S, M, flat2)
    ids2d = items32.reshape(_NIDS // 128, 128)
    gc, gr = gather_emb(W_center, W_radius, ids2d)
    F = _SET_LEN * _DIM
    gc2 = gc.reshape(_NSETS_B, F)
    gr2 = gr.reshape(_NSETS_B, F)
    ovt = jnp.transpose(overlaps)
    lanes = jnp.arange(F, dtype=jnp.int32)
    grp = jnp.arange(_SET_LEN, dtype=jnp.int32)
    E = ((lanes[:, None] // _DIM) == grp[None, :]).astype(jnp.float32)
    D = ((lanes[:, None] % _DIM) == grp[None, :]).astype(jnp.float32)
    ac8 = jnp.broadcast_to(jnp.tile(A_center, _SET_LEN), (8, F))
    ar8 = jnp.broadcast_to(jnp.tile(A_radius, _SET_LEN), (8, F))

    parts = pl.pallas_call(
        _attn_loss,
        grid=(_NBLK,),
        in_specs=[
            pl.BlockSpec((_P, F), lambda i: (i, 0)),
            pl.BlockSpec((_P, F), lambda i: (i + _NBLK, 0)),
            pl.BlockSpec((_P, F), lambda i: (i, 0)),
            pl.BlockSpec((_P, F), lambda i: (i + _NBLK, 0)),
            pl.BlockSpec((_P, _SET_LEN), lambda i: (i, 0)),
            pl.BlockSpec((_P, _SET_LEN), lambda i: (i + _NBLK, 0)),
            pl.BlockSpec((_P, 4), lambda i: (i, 0)),
            pl.BlockSpec((8, F), lambda i: (0, 0)),
            pl.BlockSpec((8, F), lambda i: (0, 0)),
            pl.BlockSpec((F, _SET_LEN), lambda i: (0, 0)),
            pl.BlockSpec((F, _SET_LEN), lambda i: (0, 0)),
            pl.BlockSpec((_SET_LEN, F), lambda i: (0, 0)),
            pl.BlockSpec((_SET_LEN, F), lambda i: (0, 0)),
        ],
        out_specs=pl.BlockSpec((1, 1, 4), lambda i: (i, 0, 0)),
        out_shape=jax.ShapeDtypeStruct((_NBLK, 1, 4), jnp.float32),
    )(gc2, gc2, gr2, gr2, mb, mb, ovt, ac8, ar8,
      E, D, jnp.transpose(E), jnp.transpose(D))

    losses = jnp.sum(parts, axis=(0, 1))
    return (losses[0], losses[1], losses[2], losses[3])


# speedup vs baseline: 66.0674x; 1.0300x over previous
import functools

import jax
import jax.numpy as jnp
from jax import lax
from jax.experimental import pallas as pl
from jax.experimental.pallas import tpu as pltpu
from jax.experimental.pallas import tpu_sc as plsc

_EPS = 1e-10
_DIM = 32
_SET_LEN = 32
_NPAIR = 16384
_NSETS_B = 2 * _NPAIR
_NIDS = _NSETS_B * _SET_LEN
_NW = 32
_P = 512
_NBLK = _NPAIR // _P

@functools.cache
def _sc_kernels():
    mesh = plsc.VectorSubcoreMesh(core_axis_name="c", subcore_axis_name="s")

    @functools.partial(
        pl.kernel,
        mesh=mesh,
        out_type=[
            jax.ShapeDtypeStruct((_NSETS_B, _SET_LEN), jnp.int32),
            jax.ShapeDtypeStruct((_NSETS_B, _SET_LEN), jnp.float32),
        ],
        scratch_types=[
            pltpu.VMEM((8, 128), jnp.int32),
            pltpu.VMEM((128, _SET_LEN), jnp.int32),
            pltpu.VMEM((128, _SET_LEN), jnp.float32),
            pltpu.SemaphoreType.DMA,
        ],
        compiler_params=pltpu.CompilerParams(use_tc_tiling_on_sc=False),
    )
    def _gather_sets(s_hbm, m_hbm, idx_hbm, items_out, mb_out, idx_v, s_v, m_v, sem):
        wid = lax.axis_index("s") * 2 + lax.axis_index("c")
        pltpu.sync_copy(idx_hbm.at[pl.ds(wid * 8, 8)], idx_v)
        for j in range(8):
            pltpu.async_copy(s_hbm.at[idx_v.at[j]], s_v, sem).wait()
            pltpu.sync_copy(s_v, items_out.at[pl.ds(wid * 1024 + j * 128, 128)])
            pltpu.async_copy(m_hbm.at[idx_v.at[j]], m_v, sem).wait()
            pltpu.sync_copy(m_v, mb_out.at[pl.ds(wid * 1024 + j * 128, 128)])

    @functools.partial(
        pl.kernel,
        mesh=mesh,
        out_type=[
            jax.ShapeDtypeStruct((_NIDS, _DIM), jnp.float32),
            jax.ShapeDtypeStruct((_NIDS, _DIM), jnp.float32),
        ],
        scratch_types=[
            pltpu.VMEM((4, 128), jnp.int32),
            pltpu.VMEM((512, _DIM), jnp.float32),
            pltpu.VMEM((512, _DIM), jnp.float32),
            pltpu.VMEM((512, _DIM), jnp.float32),
            pltpu.VMEM((512, _DIM), jnp.float32),
            pltpu.SemaphoreType.DMA,
            pltpu.SemaphoreType.DMA,
            pltpu.SemaphoreType.DMA,
        ],
        compiler_params=pltpu.CompilerParams(use_tc_tiling_on_sc=False),
    )
    def _gather_emb(wc_hbm, wr_hbm, ids_hbm, gc_out, gr_out,
                    idx_v, c_v, r_v, c2_v, r2_v, sem, semw, semw2):
        wid = lax.axis_index("s") * 2 + lax.axis_index("c")

        def chunk(c, cbuf, rbuf, first, semw):
            row0 = pl.multiple_of(wid * 256 + c * 4, 4)
            out0 = pl.multiple_of(wid * 32768 + c * 512, 512)
            pltpu.sync_copy(ids_hbm.at[pl.ds(row0, 4)], idx_v)
            @pl.when(jnp.logical_not(first))
            def _():
                pltpu.make_async_copy(cbuf, gc_out.at[pl.ds(out0, 512)], semw).wait()
                pltpu.make_async_copy(rbuf, gr_out.at[pl.ds(out0, 512)], semw).wait()

            cps = []
            for j in range(4):
                cps.append(
                    pltpu.async_copy(
                        wc_hbm.at[idx_v.at[j]], cbuf.at[pl.ds(j * 128, 128)], sem
                    )
                )
                cps.append(
                    pltpu.async_copy(
                        wr_hbm.at[idx_v.at[j]], rbuf.at[pl.ds(j * 128, 128)], sem
                    )
                )
            for cp in cps:
                cp.wait()
            pltpu.async_copy(cbuf, gc_out.at[pl.ds(out0, 512)], semw)
            pltpu.async_copy(rbuf, gr_out.at[pl.ds(out0, 512)], semw)

        def body(g, carry):
            chunk(2 * g, c_v, r_v, g == 0, semw)
            chunk(2 * g + 1, c2_v, r2_v, g == 0, semw2)
            return carry

        lax.fori_loop(0, 32, body, 0)
        pltpu.make_async_copy(c_v, gc_out.at[pl.ds(0, 512)], semw).wait()
        pltpu.make_async_copy(r_v, gr_out.at[pl.ds(0, 512)], semw).wait()
        pltpu.make_async_copy(c2_v, gc_out.at[pl.ds(0, 512)], semw2).wait()
        pltpu.make_async_copy(r2_v, gr_out.at[pl.ds(0, 512)], semw2).wait()

    return _gather_sets, _gather_emb


def _softplus(x):
    return jnp.maximum(x, 0.0) + jnp.log(1.0 + jnp.exp(-jnp.abs(x)))


def _attn_loss(xci_ref, xcj_ref, xri_ref, xrj_ref, mbi_ref, mbj_ref,
               ov_ref, ac_ref, ar_ref, e_ref, d_ref, et_ref, dt_ref, out_ref):
    mb_i = mbi_ref[...]
    mb_j = mbj_ref[...]
    E = e_ref[...]
    D = d_ref[...]
    ET = et_ref[...]
    DT = dt_ref[...]
    ac = ac_ref[0:1, :]
    ar = ar_ref[0:1, :]

    def mm(a, b):
        return jax.lax.dot_general(
            a, b, (((1,), (0,)), ((), ())),
            preferred_element_type=jnp.float32)

    x = jnp.concatenate(
        [xci_ref[...] * ac, xcj_ref[...] * ac,
         xri_ref[...] * ar, xrj_ref[...] * ar], axis=0)
    xr = jnp.concatenate(
        [xci_ref[...], xcj_ref[...], xri_ref[...], xrj_ref[...]], axis=0)
    mask = jnp.concatenate([mb_i, mb_j, mb_i, mb_j], axis=0) > 0.0

    def segsoft(v):
        vm = jnp.where(mask, v, -jnp.inf)
        m = jnp.max(vm, axis=1, keepdims=True)
        w = jnp.where(mask, jnp.exp(v - m), 0.0)
        d = jnp.sum(w, axis=1, keepdims=True)
        return w / d

    att = mm(x, E)
    w = segsoft(att)
    a = mm(xr * mm(w, ET), D)
    att2 = mm(xr * mm(a, DT), E)
    w2 = segsoft(att2)
    emb = mm(xr * mm(w2, ET), D)

    c_i = emb[0 * _P:1 * _P]
    c_j = emb[1 * _P:2 * _P]
    sz_i = jnp.exp(jnp.log(jnp.sum(mb_i, axis=1, keepdims=True)) * (1.0 / _DIM))
    sz_j = jnp.exp(jnp.log(jnp.sum(mb_j, axis=1, keepdims=True)) * (1.0 / _DIM))
    r_i = emb[2 * _P:3 * _P] * sz_i
    r_j = emb[3 * _P:4 * _P] * sz_j

    m_i = _softplus(c_i)
    be_i = _softplus(r_i)
    Mi = m_i + be_i
    m_j = _softplus(c_j)
    be_j = _softplus(r_j)
    Mj = m_j + be_j
    delta = jnp.minimum(Mi, Mj) - jnp.maximum(m_i, m_j)
    bv_i = jnp.sum(jnp.log(be_i + _EPS), axis=1, keepdims=True)
    bv_j = jnp.sum(jnp.log(be_j + _EPS), axis=1, keepdims=True)
    inter = jnp.sum(jnp.log(delta + _EPS), axis=1, keepdims=True)
    union = jnp.sum(
        jnp.log(jnp.maximum(Mi, Mj) - jnp.minimum(m_i, m_j) + _EPS),
        axis=1, keepdims=True)
    c_overlap = inter
    c_jaccard = inter / union
    denom = jnp.log(jnp.abs(bv_i) + _EPS) + jnp.log(jnp.abs(bv_j) + _EPS)
    c_cosine = inter / jnp.exp(jnp.log(denom) * (1.0 / _DIM))
    c_dice = 2.0 * inter / (bv_i + bv_j)

    ov = ov_ref[...]
    d1 = jnp.exp(c_overlap) - ov[:, 0:1]
    d2 = jnp.exp(c_jaccard) - ov[:, 1:2]
    d3 = jnp.exp(c_cosine) - ov[:, 2:3]
    d4 = jnp.exp(c_dice) - ov[:, 3:4]
    parts = jnp.concatenate(
        [jnp.sum(d1 * d1, axis=0, keepdims=True),
         jnp.sum(d2 * d2, axis=0, keepdims=True),
         jnp.sum(d3 * d3, axis=0, keepdims=True),
         jnp.sum(d4 * d4, axis=0, keepdims=True)], axis=1)
    out_ref[...] = parts.reshape(1, 1, 4)


def kernel(S, M, instances, overlaps, W_center, W_radius, A_center, A_radius):
    gather_sets, gather_emb = _sc_kernels()
    flat2 = jnp.transpose(instances).reshape(_NSETS_B // 128, 128)
    items32, mb = gather_sets(S, M, flat2)
    ids2d = items32.reshape(_NIDS // 128, 128)
    gc, gr = gather_emb(W_center, W_radius, ids2d)
    F = _SET_LEN * _DIM
    gc2 = gc.reshape(_NSETS_B, F)
    gr2 = gr.reshape(_NSETS_B, F)
    ovt = jnp.transpose(overlaps)
    lanes = jnp.arange(F, dtype=jnp.int32)
    grp = jnp.arange(_SET_LEN, dtype=jnp.int32)
    E = ((lanes[:, None] // _DIM) == grp[None, :]).astype(jnp.float32)
    D = ((lanes[:, None] % _DIM) == grp[None, :]).astype(jnp.float32)
    ac8 = jnp.broadcast_to(jnp.tile(A_center, _SET_LEN), (8, F))
    ar8 = jnp.broadcast_to(jnp.tile(A_radius, _SET_LEN), (8, F))

    parts = pl.pallas_call(
        _attn_loss,
        grid=(_NBLK,),
        in_specs=[
            pl.BlockSpec((_P, F), lambda i: (i, 0)),
            pl.BlockSpec((_P, F), lambda i: (i + _NBLK, 0)),
            pl.BlockSpec((_P, F), lambda i: (i, 0)),
            pl.BlockSpec((_P, F), lambda i: (i + _NBLK, 0)),
            pl.BlockSpec((_P, _SET_LEN), lambda i: (i, 0)),
            pl.BlockSpec((_P, _SET_LEN), lambda i: (i + _NBLK, 0)),
            pl.BlockSpec((_P, 4), lambda i: (i, 0)),
            pl.BlockSpec((8, F), lambda i: (0, 0)),
            pl.BlockSpec((8, F), lambda i: (0, 0)),
            pl.BlockSpec((F, _SET_LEN), lambda i: (0, 0)),
            pl.BlockSpec((F, _SET_LEN), lambda i: (0, 0)),
            pl.BlockSpec((_SET_LEN, F), lambda i: (0, 0)),
            pl.BlockSpec((_SET_LEN, F), lambda i: (0, 0)),
        ],
        out_specs=pl.BlockSpec((1, 1, 4), lambda i: (i, 0, 0)),
        out_shape=jax.ShapeDtypeStruct((_NBLK, 1, 4), jnp.float32),
    )(gc2, gc2, gr2, gr2, mb, mb, ovt, ac8, ar8,
      E, D, jnp.transpose(E), jnp.transpose(D))

    losses = jnp.sum(parts, axis=(0, 1))
    return (losses[0], losses[1], losses[2], losses[3])
